# baseline (speedup 1.0000x reference)
import jax
import jax.numpy as jnp
import numpy as np
from jax import lax
from jax.experimental import pallas as pl
from jax.experimental.pallas import tpu as pltpu

_GL_X, _GL_W = np.polynomial.legendre.leggauss(4)
_GL_K = tuple(float(v) for v in (0.5 * (_GL_X + 1.0)))
_GL_WH = tuple(float(v) for v in (0.5 * _GL_W))
_INV_TWO_PI = 0.15915494309189535
_LOG2E = 1.4426950408889634
_NQ = 4
_CT = 1024
_BR = 512
_CH = 8

_SQRT2 = 1.4142135623730951
_ERFINV_W_COEFFS = [_SQRT2 * c for c in (
    4.195203037562853e-05, -0.00011155266490761961,
    -0.0023518462548096832, 0.011556204278438498,
    0.23201268824921592, 0.8862269473593245)]
_P_W = np.polynomial.Polynomial(_ERFINV_W_COEFFS[::-1])
_P_V = _P_W(np.polynomial.Polynomial([-np.log(4.0), -1.0]))
_ERFINV_V_COEFFS = tuple(float(c) for c in _P_V.coef[::-1])


def _ndtr(x):
    return 0.5 * (1.0 + lax.erf(x * jnp.float32(0.7071067811865476)))


def _s_recip(x):
    return (1.0 / jnp.full((1, 128), x, jnp.float32))[0, 0]


def _s_rsqrt(x):
    return lax.rsqrt(jnp.full((1, 128), x, jnp.float32))[0, 0]


def _s_log2(x):
    return jnp.log2(jnp.full((1, 128), x, jnp.float32))[0, 0]


def _loss_block(g12_ref, g34_ref, g3412_ref, s1_ref, s2_ref,
                yh_ref, y_ref, out_ref):
    a = g12_ref[0, 0]
    b = g12_ref[0, 1]
    c = g12_ref[1, 0]
    d = g12_ref[1, 1]
    rdet = _s_recip(a * d - b * c)
    i00 = d * rdet
    i01 = -b * rdet
    i10 = -c * rdet
    i11 = a * rdet
    g0 = g3412_ref[0, 0]
    g1 = g3412_ref[0, 1]
    g2 = g3412_ref[1, 0]
    g3 = g3412_ref[1, 1]
    a00 = g0 * i00 + g1 * i10
    a01 = g0 * i01 + g1 * i11
    a10 = g2 * i00 + g3 * i10
    a11 = g2 * i01 + g3 * i11
    s00 = g34_ref[0, 0] - (a00 * g0 + a01 * g1)
    s01 = g34_ref[0, 1] - (a00 * g2 + a01 * g3)
    s11 = g34_ref[1, 1] - (a10 * g2 + a11 * g3)
    i01s = i01 + i10
    inv_s1 = _s_recip(s1_ref[0])
    inv_s2 = _s_recip(s2_ref[0])
    inv_s1g = _s_rsqrt(s00)
    inv_s2g = _s_rsqrt(s11)
    rho = s01 * inv_s1g * inv_s2g
    m00 = a00 * inv_s1 * inv_s1g
    m01 = a01 * inv_s2 * inv_s1g
    m10 = a10 * inv_s1 * inv_s2g
    m11 = a11 * inv_s2 * inv_s2g
    qc1 = 0.5 * i00 * inv_s1 * inv_s1
    qc2 = 0.5 * i01s * inv_s1 * inv_s2
    qc3 = 0.5 * i11 * inv_s2 * inv_s2
    c3s = [x * inv_s1g for x in [jnp.float32(c) for c in _ERFINV_V_COEFFS]]
    c4s = [x * inv_s2g for x in [jnp.float32(c) for c in _ERFINV_V_COEFFS]]
    half_log2e = jnp.float32(0.5 * _LOG2E)
    aqs, bqs, cqs = [], [], []
    for q in range(_NQ):
        r_q = rho * jnp.float32(_GL_K[q])
        rom = _s_rsqrt(1.0 - r_q * r_q)
        rom2 = rom * rom
        aqs.append(half_log2e * rom2)
        bqs.append(jnp.float32(_LOG2E) * r_q * rom2)
        dq = jnp.float32(_GL_WH[q] * _INV_TWO_PI) * rho * rom
        cqs.append(_s_log2(dq))

    total = None
    for rr in range(0, _BR, _CH):
        sl = slice(rr, rr + _CH)
        p3 = yh_ref[0, sl, :]
        m1 = yh_ref[1, sl, :]
        p4 = yh_ref[2, sl, :]
        m2 = yh_ref[3, sl, :]
        l3 = y_ref[0, sl, :]
        r1 = y_ref[1, sl, :]
        l4 = y_ref[2, sl, :]
        r2 = y_ref[3, sl, :]

        e1r = r1 - m1
        e2r = r2 - m2
        mu1s = m00 * e1r + m01 * e2r
        mu2s = m10 * e1r + m11 * e2r
        quad_half = (qc1 * e1r + qc2 * e2r) * e1r + qc3 * e2r * e2r

        om3 = 1.0 - p3
        om4 = 1.0 - p4
        v3 = jnp.log(p3 * om3)
        v4 = jnp.log(p4 * om4)
        x3 = om3 - p3
        x4 = om4 - p4
        t3s = c3s[0]
        t4s = c4s[0]
        for i in range(1, len(c3s)):
            t3s = t3s * v3 + c3s[i]
            t4s = t4s * v4 + c4s[i]
        h = t3s * x3 - mu1s
        k = t4s * x4 - mu2s
        p3n = _ndtr(h)
        p4n = _ndtr(k)

        s = h * h + k * k
        hk = h * k
        acc = p3n * p4n
        for q in range(_NQ):
            acc = acc + jnp.exp2(hk * bqs[q] + (cqs[q] - s * aqs[q]))

        base = jnp.where(l3 < 1.0,
                         jnp.where(l4 < 1.0, 0.0, p3n),
                         jnp.where(l4 < 1.0, p4n, 1.0 - p3n - p4n))
        sign = (1.0 - 2.0 * l3) * (1.0 - 2.0 * l4)
        ci = base + sign * acc
        log_ci = jnp.log(jnp.maximum(ci, 1e-30))
        part = jnp.sum(quad_half - log_ci, keepdims=True)
        total = part if total is None else total + part
    out_ref[0] = total


def kernel(y_hat, y, gamma12, gamma34, gamma3412, sigma1, sigma2):
    f32 = jnp.float32
    n = y_hat.shape[1]
    rows = n // _CT
    grid = rows // _BR

    yh3 = y_hat.reshape(4, rows, _CT)
    y3 = y.reshape(4, rows, _CT)

    smem = pl.BlockSpec(memory_space=pltpu.SMEM)
    partials = pl.pallas_call(
        _loss_block,
        grid=(grid,),
        in_specs=[
            smem, smem, smem, smem, smem,
            pl.BlockSpec((4, _BR, _CT), lambda i: (0, i, 0)),
            pl.BlockSpec((4, _BR, _CT), lambda i: (0, i, 0)),
        ],
        out_specs=pl.BlockSpec((1, 1, 1), lambda i: (i, 0, 0)),
        out_shape=jax.ShapeDtypeStruct((grid, 1, 1), f32),
        compiler_params=pltpu.CompilerParams(dimension_semantics=("parallel",)),
    )(gamma12, gamma34, gamma3412, sigma1, sigma2, yh3, y3)
    return jnp.sum(partials)


# speedup vs baseline: 1.0663x; 1.0663x over previous
import jax
import jax.numpy as jnp
import numpy as np
from jax import lax
from jax.experimental import pallas as pl
from jax.experimental.pallas import tpu as pltpu

_GL_X, _GL_W = np.polynomial.legendre.leggauss(3)
_GL_K = tuple(float(v) for v in (0.5 * (_GL_X + 1.0)))
_GL_WH = tuple(float(v) for v in (0.5 * _GL_W))
_INV_TWO_PI = 0.15915494309189535
_LOG2E = 1.4426950408889634
_NQ = 3
_CT = 1024
_BR = 256
_CH = 8

_SQRT2 = 1.4142135623730951
_ERFINV_W_COEFFS = [_SQRT2 * c for c in (
    4.195203037562853e-05, -0.00011155266490761961,
    -0.0023518462548096832, 0.011556204278438498,
    0.23201268824921592, 0.8862269473593245)]
_P_W = np.polynomial.Polynomial(_ERFINV_W_COEFFS[::-1])
_P_V = _P_W(np.polynomial.Polynomial([-np.log(4.0), -1.0]))
_ERFINV_V_COEFFS = tuple(float(c) for c in _P_V.coef[::-1])


def _ndtr(x):
    return 0.5 * (1.0 + lax.erf(x * jnp.float32(0.7071067811865476)))


def _s_recip(x):
    return (1.0 / jnp.full((1, 128), x, jnp.float32))[0, 0]


def _s_rsqrt(x):
    return lax.rsqrt(jnp.full((1, 128), x, jnp.float32))[0, 0]


def _s_log2(x):
    return jnp.log2(jnp.full((1, 128), x, jnp.float32))[0, 0]


def _loss_block(g12_ref, g34_ref, g3412_ref, s1_ref, s2_ref,
                yh_ref, y_ref, out_ref):
    a = g12_ref[0, 0]
    b = g12_ref[0, 1]
    c = g12_ref[1, 0]
    d = g12_ref[1, 1]
    rdet = _s_recip(a * d - b * c)
    i00 = d * rdet
    i01 = -b * rdet
    i10 = -c * rdet
    i11 = a * rdet
    g0 = g3412_ref[0, 0]
    g1 = g3412_ref[0, 1]
    g2 = g3412_ref[1, 0]
    g3 = g3412_ref[1, 1]
    a00 = g0 * i00 + g1 * i10
    a01 = g0 * i01 + g1 * i11
    a10 = g2 * i00 + g3 * i10
    a11 = g2 * i01 + g3 * i11
    s00 = g34_ref[0, 0] - (a00 * g0 + a01 * g1)
    s01 = g34_ref[0, 1] - (a00 * g2 + a01 * g3)
    s11 = g34_ref[1, 1] - (a10 * g2 + a11 * g3)
    i01s = i01 + i10
    inv_s1 = _s_recip(s1_ref[0])
    inv_s2 = _s_recip(s2_ref[0])
    inv_s1g = _s_rsqrt(s00)
    inv_s2g = _s_rsqrt(s11)
    rho = s01 * inv_s1g * inv_s2g
    m00 = a00 * inv_s1 * inv_s1g
    m01 = a01 * inv_s2 * inv_s1g
    m10 = a10 * inv_s1 * inv_s2g
    m11 = a11 * inv_s2 * inv_s2g
    qc1 = 0.5 * i00 * inv_s1 * inv_s1
    qc2 = 0.5 * i01s * inv_s1 * inv_s2
    qc3 = 0.5 * i11 * inv_s2 * inv_s2
    c3s = [x * inv_s1g for x in [jnp.float32(c) for c in _ERFINV_V_COEFFS]]
    c4s = [x * inv_s2g for x in [jnp.float32(c) for c in _ERFINV_V_COEFFS]]
    half_log2e = jnp.float32(0.5 * _LOG2E)
    aqs, bqs, cqs = [], [], []
    for q in range(_NQ):
        r_q = rho * jnp.float32(_GL_K[q])
        rom = _s_rsqrt(1.0 - r_q * r_q)
        rom2 = rom * rom
        aqs.append(half_log2e * rom2)
        bqs.append(jnp.float32(_LOG2E) * r_q * rom2)
        dq = jnp.float32(_GL_WH[q] * _INV_TWO_PI) * rho * rom
        cqs.append(_s_log2(dq))

    total = None
    for rr in range(0, _BR, _CH):
        sl = slice(rr, rr + _CH)
        p3 = yh_ref[0, sl, :]
        m1 = yh_ref[1, sl, :]
        p4 = yh_ref[2, sl, :]
        m2 = yh_ref[3, sl, :]
        l3 = y_ref[0, sl, :]
        r1 = y_ref[1, sl, :]
        l4 = y_ref[2, sl, :]
        r2 = y_ref[3, sl, :]

        e1r = r1 - m1
        e2r = r2 - m2
        mu1s = m00 * e1r + m01 * e2r
        mu2s = m10 * e1r + m11 * e2r
        quad_half = (qc1 * e1r + qc2 * e2r) * e1r + qc3 * e2r * e2r

        om3 = 1.0 - p3
        om4 = 1.0 - p4
        v3 = jnp.log(p3 * om3)
        v4 = jnp.log(p4 * om4)
        x3 = om3 - p3
        x4 = om4 - p4
        t3s = c3s[0]
        t4s = c4s[0]
        for i in range(1, len(c3s)):
            t3s = t3s * v3 + c3s[i]
            t4s = t4s * v4 + c4s[i]
        h = t3s * x3 - mu1s
        k = t4s * x4 - mu2s
        p3n = _ndtr(h)
        p4n = _ndtr(k)

        s = h * h + k * k
        hk = h * k
        dcorr = jnp.exp2(hk * bqs[0] + (cqs[0] - s * aqs[0]))
        for q in range(1, _NQ):
            dcorr = dcorr + jnp.exp2(hk * bqs[q] + (cqs[q] - s * aqs[q]))

        s3 = 1.0 - 2.0 * l3
        s4 = 1.0 - 2.0 * l4
        q3 = s3 * p3n + l3
        q4 = s4 * p4n + l4
        ci = q3 * q4 + (s3 * s4) * dcorr
        log_ci = jnp.log(jnp.maximum(ci, 1e-30))
        part = jnp.sum(quad_half - log_ci, keepdims=True)
        total = part if total is None else total + part
    out_ref[0] = total


def kernel(y_hat, y, gamma12, gamma34, gamma3412, sigma1, sigma2):
    f32 = jnp.float32
    n = y_hat.shape[1]
    rows = n // _CT
    grid = rows // _BR

    yh3 = y_hat.reshape(4, rows, _CT)
    y3 = y.reshape(4, rows, _CT)

    smem = pl.BlockSpec(memory_space=pltpu.SMEM)
    partials = pl.pallas_call(
        _loss_block,
        grid=(grid,),
        in_specs=[
            smem, smem, smem, smem, smem,
            pl.BlockSpec((4, _BR, _CT), lambda i: (0, i, 0)),
            pl.BlockSpec((4, _BR, _CT), lambda i: (0, i, 0)),
        ],
        out_specs=pl.BlockSpec((1, 1, 1), lambda i: (i, 0, 0)),
        out_shape=jax.ShapeDtypeStruct((grid, 1, 1), f32),
        compiler_params=pltpu.CompilerParams(dimension_semantics=("parallel",)),
    )(gamma12, gamma34, gamma3412, sigma1, sigma2, yh3, y3)
    return jnp.sum(partials)


# speedup vs baseline: 1.0671x; 1.0008x over previous
import jax
import jax.numpy as jnp
import numpy as np
from jax import lax
from jax.experimental import pallas as pl
from jax.experimental.pallas import tpu as pltpu

_GL_X, _GL_W = np.polynomial.legendre.leggauss(3)
_GL_K = tuple(float(v) for v in (0.5 * (_GL_X + 1.0)))
_GL_WH = tuple(float(v) for v in (0.5 * _GL_W))
_INV_TWO_PI = 0.15915494309189535
_LOG2E = 1.4426950408889634
_NQ = 3
_CT = 1024
_BR = 256
_CH = 8

_SQRT2 = 1.4142135623730951
_ERFINV_W_COEFFS = [_SQRT2 * c for c in (
    4.195203037562853e-05, -0.00011155266490761961,
    -0.0023518462548096832, 0.011556204278438498,
    0.23201268824921592, 0.8862269473593245)]
_P_W = np.polynomial.Polynomial(_ERFINV_W_COEFFS[::-1])
_P_V = _P_W(np.polynomial.Polynomial([-np.log(4.0), -1.0]))
_ERFINV_V_COEFFS = tuple(float(c) for c in _P_V.coef[::-1])


def _ndtr(x):
    return 0.5 * (1.0 + lax.erf(x * jnp.float32(0.7071067811865476)))


def _s_recip(x):
    return (1.0 / jnp.full((1, 128), x, jnp.float32))[0, 0]


def _s_rsqrt(x):
    return lax.rsqrt(jnp.full((1, 128), x, jnp.float32))[0, 0]


def _s_log2(x):
    return jnp.log2(jnp.full((1, 128), x, jnp.float32))[0, 0]


def _loss_block(g12_ref, g34_ref, g3412_ref, s1_ref, s2_ref,
                yh_ref, y_ref, out_ref):
    a = g12_ref[0, 0]
    b = g12_ref[0, 1]
    c = g12_ref[1, 0]
    d = g12_ref[1, 1]
    rdet = _s_recip(a * d - b * c)
    i00 = d * rdet
    i01 = -b * rdet
    i10 = -c * rdet
    i11 = a * rdet
    g0 = g3412_ref[0, 0]
    g1 = g3412_ref[0, 1]
    g2 = g3412_ref[1, 0]
    g3 = g3412_ref[1, 1]
    a00 = g0 * i00 + g1 * i10
    a01 = g0 * i01 + g1 * i11
    a10 = g2 * i00 + g3 * i10
    a11 = g2 * i01 + g3 * i11
    s00 = g34_ref[0, 0] - (a00 * g0 + a01 * g1)
    s01 = g34_ref[0, 1] - (a00 * g2 + a01 * g3)
    s11 = g34_ref[1, 1] - (a10 * g2 + a11 * g3)
    i01s = i01 + i10
    inv_s1 = _s_recip(s1_ref[0])
    inv_s2 = _s_recip(s2_ref[0])
    inv_s1g = _s_rsqrt(s00)
    inv_s2g = _s_rsqrt(s11)
    rho = s01 * inv_s1g * inv_s2g
    m00 = a00 * inv_s1 * inv_s1g
    m01 = a01 * inv_s2 * inv_s1g
    m10 = a10 * inv_s1 * inv_s2g
    m11 = a11 * inv_s2 * inv_s2g
    qc1 = 0.5 * i00 * inv_s1 * inv_s1
    qc2 = 0.5 * i01s * inv_s1 * inv_s2
    qc3 = 0.5 * i11 * inv_s2 * inv_s2
    c3s = [x * inv_s1g for x in [jnp.float32(c) for c in _ERFINV_V_COEFFS]]
    c4s = [x * inv_s2g for x in [jnp.float32(c) for c in _ERFINV_V_COEFFS]]
    half_log2e = jnp.float32(0.5 * _LOG2E)
    aqs, bqs, cqs = [], [], []
    for q in range(_NQ):
        r_q = rho * jnp.float32(_GL_K[q])
        rom = _s_rsqrt(1.0 - r_q * r_q)
        rom2 = rom * rom
        aqs.append(half_log2e * rom2)
        bqs.append(jnp.float32(_LOG2E) * r_q * rom2)
        dq = jnp.float32(_GL_WH[q] * _INV_TWO_PI) * rho * rom
        cqs.append(_s_log2(dq))

    total = None
    for rr in range(0, _BR, _CH):
        sl = slice(rr, rr + _CH)
        p3 = yh_ref[0, sl, :]
        m1 = yh_ref[1, sl, :]
        p4 = yh_ref[2, sl, :]
        m2 = yh_ref[3, sl, :]
        l3 = y_ref[0, sl, :]
        r1 = y_ref[1, sl, :]
        l4 = y_ref[2, sl, :]
        r2 = y_ref[3, sl, :]

        e1r = r1 - m1
        e2r = r2 - m2
        mu1s = m00 * e1r + m01 * e2r
        mu2s = m10 * e1r + m11 * e2r
        quad_half = (qc1 * e1r + qc2 * e2r) * e1r + qc3 * e2r * e2r

        om3 = 1.0 - p3
        om4 = 1.0 - p4
        v3 = jnp.log(p3 * om3)
        v4 = jnp.log(p4 * om4)
        x3 = om3 - p3
        x4 = om4 - p4
        t3s = c3s[0]
        t4s = c4s[0]
        for i in range(1, len(c3s)):
            t3s = t3s * v3 + c3s[i]
            t4s = t4s * v4 + c4s[i]
        h = t3s * x3 - mu1s
        k = t4s * x4 - mu2s
        p3n = _ndtr(h)
        p4n = _ndtr(k)

        s = h * h + k * k
        hk = h * k
        dcorr = jnp.exp2(hk * bqs[0] + (cqs[0] - s * aqs[0]))
        for q in range(1, _NQ):
            dcorr = dcorr + jnp.exp2(hk * bqs[q] + (cqs[q] - s * aqs[q]))

        s3 = 1.0 - 2.0 * l3
        s4 = 1.0 - 2.0 * l4
        q3 = s3 * p3n + l3
        q4 = s4 * p4n + l4
        ci = q3 * q4 + (s3 * s4) * dcorr
        log_ci = jnp.log(jnp.maximum(ci, 1e-30))
        term = quad_half - log_ci
        total = term if total is None else total + term
    out_ref[0] = jnp.sum(total, keepdims=True)


def kernel(y_hat, y, gamma12, gamma34, gamma3412, sigma1, sigma2):
    f32 = jnp.float32
    n = y_hat.shape[1]
    rows = n // _CT
    grid = rows // _BR

    yh3 = y_hat.reshape(4, rows, _CT)
    y3 = y.reshape(4, rows, _CT)

    smem = pl.BlockSpec(memory_space=pltpu.SMEM)
    partials = pl.pallas_call(
        _loss_block,
        grid=(grid,),
        in_specs=[
            smem, smem, smem, smem, smem,
            pl.BlockSpec((4, _BR, _CT), lambda i: (0, i, 0)),
            pl.BlockSpec((4, _BR, _CT), lambda i: (0, i, 0)),
        ],
        out_specs=pl.BlockSpec((1, 1, 1), lambda i: (i, 0, 0)),
        out_shape=jax.ShapeDtypeStruct((grid, 1, 1), f32),
        compiler_params=pltpu.CompilerParams(dimension_semantics=("parallel",)),
    )(gamma12, gamma34, gamma3412, sigma1, sigma2, yh3, y3)
    return jnp.sum(partials)
